# baseline (speedup 1.0000x reference)
import functools

import jax
import jax.numpy as jnp
from jax import lax
from jax.experimental import pallas as pl
from jax.experimental.pallas import tpu as pltpu
from jax.experimental.pallas import tpu_sc as plsc

N = 50000
E = 800000
EP = 819200
NBLK = EP // 128
BPT = NBLK // 16
GRP = 20
NGRP = BPT // GRP
ROWS = 50048
RPT = ROWS // 16
CROWS = 50176
CPT = CROWS // 16
DUMMY = N

_f32 = jnp.float32
_mesh = plsc.VectorSubcoreMesh(core_axis_name="c", subcore_axis_name="s")
_sc_params = pltpu.CompilerParams(use_tc_tiling_on_sc=False)


_NR = 5


def _one_conv(g_both, src2d, dst2d, z2d, out, idx_s, idx_d, rows,
              sem_g, sem_s, acc, c, s):
    base = s * RPT
    for k in range(24):
        pltpu.sync_copy(z2d, acc.at[pl.ds(base + k * 128, 128)])
    pltpu.sync_copy(z2d, acc.at[pl.ds(base + RPT - 128, 128)])
    plsc.subcore_barrier()

    rb0 = s * BPT

    def _fire_scatter(ring, slot, off, j):
        pltpu.make_async_copy(g_both.at[c].at[idx_s.at[slot].at[off]],
                              rows.at[ring], sem_g[ring]).wait()
        pltpu.async_copy(rows.at[ring], acc.at[idx_d.at[slot].at[off]],
                         sem_s[ring], add=True)

    def grp_body(gi, carry):
        slot = lax.rem(gi, 2)
        rowbase = rb0 + gi * GRP
        pltpu.sync_copy(src2d.at[pl.ds(rowbase, GRP)], idx_s.at[slot])
        pltpu.sync_copy(dst2d.at[pl.ds(rowbase, GRP)], idx_d.at[slot])
        for k in range(GRP):
            j = gi * GRP + k
            ring = k % _NR

            @pl.when(j >= _NR)
            def _():
                pltpu.make_async_copy(rows.at[ring],
                                      acc.at[idx_d.at[slot].at[k]],
                                      sem_s[ring]).wait()

            pltpu.async_copy(g_both.at[c].at[idx_s.at[slot].at[k]],
                             rows.at[ring], sem_g[ring])

            @pl.when(j >= 3)
            def _():
                ringd = (k - 3) % _NR
                offd = (k - 3) % GRP
                slotd = lax.rem(gi - (1 if k < 3 else 0), 2)
                _fire_scatter(ringd, slotd, offd, j - 3)

        return carry

    lax.fori_loop(0, NGRP, grp_body, 0)
    last_slot = (NGRP - 1) % 2
    for jd in range(BPT - 3, BPT):
        _fire_scatter(jd % _NR, last_slot, jd % GRP, jd)
    for r in range(_NR):
        pltpu.make_async_copy(rows.at[r], acc.at[idx_d.at[last_slot].at[0]],
                              sem_s[r]).wait()
    plsc.subcore_barrier()
    pltpu.sync_copy(acc.at[pl.ds(base, RPT)], out.at[c].at[pl.ds(base, RPT)])


def _conv2_body(ga, sa, da, gb, sb, db, z2d, out_a, out_b, idx_s, idx_d, rows,
                sg0, sg1, sg2, sg3, sg4, ss0, ss1, ss2, ss3, ss4, acc):
    c = lax.axis_index("c")
    s = lax.axis_index("s")
    sem_g = [sg0, sg1, sg2, sg3, sg4]
    sem_s = [ss0, ss1, ss2, ss3, ss4]
    _one_conv(ga, sa, da, z2d, out_a, idx_s, idx_d, rows, sem_g, sem_s,
              acc, c, s)
    plsc.subcore_barrier()
    _one_conv(gb, sb, db, z2d, out_b, idx_s, idx_d, rows, sem_g, sem_s,
              acc, c, s)


_conv2 = pl.kernel(
    _conv2_body,
    out_type=(jax.ShapeDtypeStruct((2, ROWS, 32), _f32),
              jax.ShapeDtypeStruct((2, ROWS, 32), _f32)),
    mesh=_mesh,
    scratch_types=[
        pltpu.VMEM((2, GRP, 128), jnp.int32),
        pltpu.VMEM((2, GRP, 128), jnp.int32),
        pltpu.VMEM((_NR, 128, 32), _f32),
    ] + [pltpu.SemaphoreType.DMA] * 10 + [
        pltpu.VMEM_SHARED((ROWS, 32), _f32),
    ],
    compiler_params=_sc_params,
)


def _count_body(dst2d_both, one_rows, z2d, out, idx_d, obuf, sem_s, cacc):
    c = lax.axis_index("c")
    s = lax.axis_index("s")
    base = s * CPT
    for k in range(24):
        pltpu.sync_copy(z2d, cacc.at[pl.ds(base + k * 128, 128)])
    pltpu.sync_copy(z2d, cacc.at[pl.ds(base + CPT - 128, 128)])
    pltpu.sync_copy(one_rows, obuf)
    plsc.subcore_barrier()

    def grp_body(gi, carry):
        slot = lax.rem(gi, 2)
        rowbase = s * BPT + gi * GRP
        pltpu.sync_copy(dst2d_both.at[c].at[pl.ds(rowbase, GRP)],
                        idx_d.at[slot])

        def blk(j, carry2):
            pltpu.sync_copy(obuf, cacc.at[idx_d.at[slot].at[j]], add=True)
            return carry2

        return lax.fori_loop(0, GRP, blk, carry)

    lax.fori_loop(0, NGRP, grp_body, 0)
    plsc.subcore_barrier()
    pltpu.sync_copy(cacc.at[pl.ds(base, CPT)], out.at[c].at[pl.ds(base, CPT)])


_count = pl.kernel(
    _count_body,
    out_type=jax.ShapeDtypeStruct((2, CROWS, 16), _f32),
    mesh=_mesh,
    scratch_types=[
        pltpu.VMEM((2, GRP, 128), jnp.int32),
        pltpu.VMEM((128, 16), _f32),
        pltpu.SemaphoreType.DMA,
        pltpu.VMEM_SHARED((CROWS, 16), _f32),
    ],
    compiler_params=_sc_params,
)


_R = 2000
_GRID = N // _R


def _proj_body(x_ref, wp, bp, wl, wr, bd, glo, ghi, r0):
    h = jnp.maximum(
        jnp.dot(x_ref[...], wp[...], preferred_element_type=_f32) + bp[...], 0.0)
    g = jnp.dot(h, wl[...], preferred_element_type=_f32)
    glo[...] = g[:, :32]
    ghi[...] = g[:, 32:]
    r0[...] = jnp.dot(h, wr[...], preferred_element_type=_f32) + bd[...] + h


def _comb_body(slo, shi, cnt, r0, wl, wr, b, glo, ghi, r1):
    sm = jnp.concatenate([slo[...], shi[...]], axis=1)
    h1 = jnp.maximum(sm / jnp.maximum(cnt[...], 1.0) + r0[...], 0.0)
    g = jnp.dot(h1, wl[...], preferred_element_type=_f32)
    glo[...] = g[:, :32]
    ghi[...] = g[:, 32:]
    r1[...] = jnp.dot(h1, wr[...], preferred_element_type=_f32) + b[...] + h1


def _final_body(slo_u, shi_u, cnt_u, r1_u, slo_i, shi_i, cnt_i, r1_i, ou, oi):
    sm_u = jnp.concatenate([slo_u[...], shi_u[...]], axis=1)
    ou[...] = sm_u / jnp.maximum(cnt_u[...], 1.0) + r1_u[...]
    sm_i = jnp.concatenate([slo_i[...], shi_i[...]], axis=1)
    oi[...] = sm_i / jnp.maximum(cnt_i[...], 1.0) + r1_i[...]


def _rspec(w):
    return pl.BlockSpec((_R, w), lambda i: (i, 0))


def _wspec(h, w):
    return pl.BlockSpec((h, w), lambda i: (0, 0))


_proj = pl.pallas_call(
    _proj_body,
    grid=(_GRID,),
    in_specs=[_rspec(128), _wspec(128, 64), _wspec(1, 64), _wspec(64, 64),
              _wspec(64, 64), _wspec(1, 64)],
    out_specs=[_rspec(32), _rspec(32), _rspec(64)],
    out_shape=[jax.ShapeDtypeStruct((N, 32), _f32),
               jax.ShapeDtypeStruct((N, 32), _f32),
               jax.ShapeDtypeStruct((N, 64), _f32)],
)

_comb = pl.pallas_call(
    _comb_body,
    grid=(_GRID,),
    in_specs=[_rspec(32), _rspec(32), _rspec(1), _rspec(64), _wspec(64, 64),
              _wspec(64, 64), _wspec(1, 64)],
    out_specs=[_rspec(32), _rspec(32), _rspec(64)],
    out_shape=[jax.ShapeDtypeStruct((N, 32), _f32),
               jax.ShapeDtypeStruct((N, 32), _f32),
               jax.ShapeDtypeStruct((N, 64), _f32)],
)

_final = pl.pallas_call(
    _final_body,
    grid=(_GRID,),
    in_specs=[_rspec(32), _rspec(32), _rspec(1), _rspec(64)] * 2,
    out_specs=[_rspec(64), _rspec(64)],
    out_shape=[jax.ShapeDtypeStruct((N, 64), _f32),
               jax.ShapeDtypeStruct((N, 64), _f32)],
)


def _pad_edges(ei):
    src = jnp.concatenate(
        [ei[0].astype(jnp.int32), jnp.zeros((EP - E,), jnp.int32)])
    dst = jnp.concatenate(
        [ei[1].astype(jnp.int32), jnp.full((EP - E,), DUMMY, jnp.int32)])
    return src.reshape(NBLK, 128), dst.reshape(NBLK, 128)


def kernel(x_user, x_item, edge_index_user_to_item, edge_index_item_to_user,
           W_proj_user, b_proj_user, W_proj_item, b_proj_item,
           Wl0_u2i, Wr0_u2i, b0_u2i, Wl0_i2u, Wr0_i2u, b0_i2u,
           Wl1_u2i, Wr1_u2i, b1_u2i, Wl1_i2u, Wr1_i2u, b1_i2u):
    src_u2i, dst_u2i = _pad_edges(edge_index_user_to_item)
    src_i2u, dst_i2u = _pad_edges(edge_index_item_to_user)

    z2d32 = jnp.zeros((128, 32), _f32)
    z2d16 = jnp.zeros((128, 16), _f32)
    one_rows = jnp.zeros((128, 16), _f32).at[:, 0].set(1.0)

    cnt_out = _count(jnp.stack([dst_u2i, dst_i2u]), one_rows, z2d16)
    cnt_item = cnt_out[0, :N, 0:1]
    cnt_user = cnt_out[1, :N, 0:1]

    def b2(v):
        return v.reshape(1, 64)

    gu_lo, gu_hi, r0_user = _proj(x_user, W_proj_user, b2(b_proj_user),
                                  Wl0_u2i, Wr0_i2u, b2(b0_i2u))
    gi_lo, gi_hi, r0_item = _proj(x_item, W_proj_item, b2(b_proj_item),
                                  Wl0_i2u, Wr0_u2i, b2(b0_u2i))

    s0_item, s0_user = _conv2(jnp.stack([gu_lo, gu_hi]), src_u2i, dst_u2i,
                              jnp.stack([gi_lo, gi_hi]), src_i2u, dst_i2u,
                              z2d32)

    g1u_lo, g1u_hi, r1_user = _comb(s0_user[0, :N], s0_user[1, :N], cnt_user,
                                    r0_user, Wl1_u2i, Wr1_i2u, b2(b1_i2u))
    g1i_lo, g1i_hi, r1_item = _comb(s0_item[0, :N], s0_item[1, :N], cnt_item,
                                    r0_item, Wl1_i2u, Wr1_u2i, b2(b1_u2i))

    s1_item, s1_user = _conv2(jnp.stack([g1u_lo, g1u_hi]), src_u2i, dst_u2i,
                              jnp.stack([g1i_lo, g1i_hi]), src_i2u, dst_i2u,
                              z2d32)

    o_user, o_item = _final(s1_user[0, :N], s1_user[1, :N], cnt_user, r1_user,
                            s1_item[0, :N], s1_item[1, :N], cnt_item, r1_item)
    return (o_user, o_item)


# speedup vs baseline: 1.0448x; 1.0448x over previous
import functools

import jax
import jax.numpy as jnp
from jax import lax
from jax.experimental import pallas as pl
from jax.experimental.pallas import tpu as pltpu
from jax.experimental.pallas import tpu_sc as plsc

N = 50000
E = 800000
EP = 819200
NBLK = EP // 128
BPT = NBLK // 16
GRP = 20
NGRP = BPT // GRP
ROWS = 50048
RPT = ROWS // 16
CROWS = 50176
CPT = CROWS // 16
DUMMY = N

_f32 = jnp.float32
_mesh = plsc.VectorSubcoreMesh(core_axis_name="c", subcore_axis_name="s")
_sc_params = pltpu.CompilerParams(use_tc_tiling_on_sc=False)


_NR = 5


def _conv_body(g_both, src2d, dst2d, z2d, out, idx_s, idx_d, rows,
               sg0, sg1, sg2, sg3, sg4, ss0, ss1, ss2, ss3, ss4, acc):
    c = lax.axis_index("c")
    s = lax.axis_index("s")
    sem_g = [sg0, sg1, sg2, sg3, sg4]
    sem_s = [ss0, ss1, ss2, ss3, ss4]
    base = s * RPT
    for k in range(24):
        pltpu.sync_copy(z2d, acc.at[pl.ds(base + k * 128, 128)])
    pltpu.sync_copy(z2d, acc.at[pl.ds(base + RPT - 128, 128)])
    plsc.subcore_barrier()

    rb0 = s * BPT

    def _fire_scatter(ring, slot, off, j):
        pltpu.make_async_copy(g_both.at[c].at[idx_s.at[slot].at[off]],
                              rows.at[ring], sem_g[ring]).wait()
        pltpu.async_copy(rows.at[ring], acc.at[idx_d.at[slot].at[off]],
                         sem_s[ring], add=True)

    def grp_body(gi, carry):
        slot = lax.rem(gi, 2)
        rowbase = rb0 + gi * GRP
        pltpu.sync_copy(src2d.at[pl.ds(rowbase, GRP)], idx_s.at[slot])
        pltpu.sync_copy(dst2d.at[pl.ds(rowbase, GRP)], idx_d.at[slot])
        for k in range(GRP):
            j = gi * GRP + k
            ring = k % _NR

            @pl.when(j >= _NR)
            def _():
                pltpu.make_async_copy(rows.at[ring],
                                      acc.at[idx_d.at[slot].at[k]],
                                      sem_s[ring]).wait()

            pltpu.async_copy(g_both.at[c].at[idx_s.at[slot].at[k]],
                             rows.at[ring], sem_g[ring])

            @pl.when(j >= 3)
            def _():
                ringd = (k - 3) % _NR
                offd = (k - 3) % GRP
                slotd = lax.rem(gi - (1 if k < 3 else 0), 2)
                _fire_scatter(ringd, slotd, offd, j - 3)

        return carry

    lax.fori_loop(0, NGRP, grp_body, 0)
    last_slot = (NGRP - 1) % 2
    for jd in range(BPT - 3, BPT):
        _fire_scatter(jd % _NR, last_slot, jd % GRP, jd)
    for r in range(_NR):
        pltpu.make_async_copy(rows.at[r], acc.at[idx_d.at[last_slot].at[0]],
                              sem_s[r]).wait()
    plsc.subcore_barrier()
    pltpu.sync_copy(acc.at[pl.ds(base, RPT)], out.at[c].at[pl.ds(base, RPT)])


_conv = pl.kernel(
    _conv_body,
    out_type=jax.ShapeDtypeStruct((2, ROWS, 32), _f32),
    mesh=_mesh,
    scratch_types=[
        pltpu.VMEM((2, GRP, 128), jnp.int32),
        pltpu.VMEM((2, GRP, 128), jnp.int32),
        pltpu.VMEM((_NR, 128, 32), _f32),
    ] + [pltpu.SemaphoreType.DMA] * 10 + [
        pltpu.VMEM_SHARED((ROWS, 32), _f32),
    ],
    compiler_params=_sc_params,
)


def _count_body(dst2d_both, one_rows, z2d, out, idx_d, obuf, cacc):
    c = lax.axis_index("c")
    s = lax.axis_index("s")
    base = s * CPT
    for k in range(24):
        pltpu.sync_copy(z2d, cacc.at[pl.ds(base + k * 128, 128)])
    pltpu.sync_copy(z2d, cacc.at[pl.ds(base + CPT - 128, 128)])
    pltpu.sync_copy(one_rows, obuf)
    plsc.subcore_barrier()

    def grp_body(gi, carry):
        rowbase = s * BPT + gi * GRP
        pltpu.sync_copy(dst2d_both.at[c].at[pl.ds(rowbase, GRP)], idx_d)

        def blk(j, carry2):
            pltpu.sync_copy(obuf, cacc.at[idx_d.at[j]], add=True)
            return carry2

        return lax.fori_loop(0, GRP, blk, carry)

    lax.fori_loop(0, NGRP, grp_body, 0)
    plsc.subcore_barrier()
    pltpu.sync_copy(cacc.at[pl.ds(base, CPT)], out.at[c].at[pl.ds(base, CPT)])


_count = pl.kernel(
    _count_body,
    out_type=jax.ShapeDtypeStruct((2, CROWS, 16), _f32),
    mesh=_mesh,
    scratch_types=[
        pltpu.VMEM((GRP, 128), jnp.int32),
        pltpu.VMEM((128, 16), _f32),
        pltpu.VMEM_SHARED((CROWS, 16), _f32),
    ],
    compiler_params=_sc_params,
)


_R = 5000
_GRID = N // _R


def _proj_body(x_ref, wp, bp, wl, wr, bd, glo, ghi, r0):
    h = jnp.maximum(
        jnp.dot(x_ref[...], wp[...], preferred_element_type=_f32) + bp[...], 0.0)
    g = jnp.dot(h, wl[...], preferred_element_type=_f32)
    glo[...] = g[:, :32]
    ghi[...] = g[:, 32:]
    r0[...] = jnp.dot(h, wr[...], preferred_element_type=_f32) + bd[...] + h


def _comb_body(slo, shi, cnt, r0, wl, wr, b, glo, ghi, r1):
    sm = jnp.concatenate([slo[...], shi[...]], axis=1)
    h1 = jnp.maximum(sm / jnp.maximum(cnt[...], 1.0) + r0[...], 0.0)
    g = jnp.dot(h1, wl[...], preferred_element_type=_f32)
    glo[...] = g[:, :32]
    ghi[...] = g[:, 32:]
    r1[...] = jnp.dot(h1, wr[...], preferred_element_type=_f32) + b[...] + h1


def _final_body(slo, shi, cnt, r1, o):
    sm = jnp.concatenate([slo[...], shi[...]], axis=1)
    o[...] = sm / jnp.maximum(cnt[...], 1.0) + r1[...]


def _rspec(w):
    return pl.BlockSpec((_R, w), lambda i: (i, 0))


def _wspec(h, w):
    return pl.BlockSpec((h, w), lambda i: (0, 0))


_proj = pl.pallas_call(
    _proj_body,
    grid=(_GRID,),
    in_specs=[_rspec(128), _wspec(128, 64), _wspec(1, 64), _wspec(64, 64),
              _wspec(64, 64), _wspec(1, 64)],
    out_specs=[_rspec(32), _rspec(32), _rspec(64)],
    out_shape=[jax.ShapeDtypeStruct((N, 32), _f32),
               jax.ShapeDtypeStruct((N, 32), _f32),
               jax.ShapeDtypeStruct((N, 64), _f32)],
)

_comb = pl.pallas_call(
    _comb_body,
    grid=(_GRID,),
    in_specs=[_rspec(32), _rspec(32), _rspec(1), _rspec(64), _wspec(64, 64),
              _wspec(64, 64), _wspec(1, 64)],
    out_specs=[_rspec(32), _rspec(32), _rspec(64)],
    out_shape=[jax.ShapeDtypeStruct((N, 32), _f32),
               jax.ShapeDtypeStruct((N, 32), _f32),
               jax.ShapeDtypeStruct((N, 64), _f32)],
)

_final = pl.pallas_call(
    _final_body,
    grid=(_GRID,),
    in_specs=[_rspec(32), _rspec(32), _rspec(1), _rspec(64)],
    out_specs=_rspec(64),
    out_shape=jax.ShapeDtypeStruct((N, 64), _f32),
)


def _pad_edges(ei):
    src = jnp.concatenate(
        [ei[0].astype(jnp.int32), jnp.zeros((EP - E,), jnp.int32)])
    dst = jnp.concatenate(
        [ei[1].astype(jnp.int32), jnp.full((EP - E,), DUMMY, jnp.int32)])
    return src.reshape(NBLK, 128), dst.reshape(NBLK, 128)


def kernel(x_user, x_item, edge_index_user_to_item, edge_index_item_to_user,
           W_proj_user, b_proj_user, W_proj_item, b_proj_item,
           Wl0_u2i, Wr0_u2i, b0_u2i, Wl0_i2u, Wr0_i2u, b0_i2u,
           Wl1_u2i, Wr1_u2i, b1_u2i, Wl1_i2u, Wr1_i2u, b1_i2u):
    src_u2i, dst_u2i = _pad_edges(edge_index_user_to_item)
    src_i2u, dst_i2u = _pad_edges(edge_index_item_to_user)

    z2d32 = jnp.zeros((128, 32), _f32)
    z2d16 = jnp.zeros((128, 16), _f32)
    one_rows = jnp.zeros((128, 16), _f32).at[:, 0].set(1.0)

    cnt_out = _count(jnp.stack([dst_u2i, dst_i2u]), one_rows, z2d16)
    cnt_item = cnt_out[0, :N, 0:1]
    cnt_user = cnt_out[1, :N, 0:1]

    def b2(v):
        return v.reshape(1, 64)

    gu_lo, gu_hi, r0_user = _proj(x_user, W_proj_user, b2(b_proj_user),
                                  Wl0_u2i, Wr0_i2u, b2(b0_i2u))
    gi_lo, gi_hi, r0_item = _proj(x_item, W_proj_item, b2(b_proj_item),
                                  Wl0_i2u, Wr0_u2i, b2(b0_u2i))

    s0_item = _conv(jnp.stack([gu_lo, gu_hi]), src_u2i, dst_u2i, z2d32)
    s0_user = _conv(jnp.stack([gi_lo, gi_hi]), src_i2u, dst_i2u, z2d32)

    g1u_lo, g1u_hi, r1_user = _comb(s0_user[0, :N], s0_user[1, :N], cnt_user,
                                    r0_user, Wl1_u2i, Wr1_i2u, b2(b1_i2u))
    g1i_lo, g1i_hi, r1_item = _comb(s0_item[0, :N], s0_item[1, :N], cnt_item,
                                    r0_item, Wl1_i2u, Wr1_u2i, b2(b1_u2i))

    s1_item = _conv(jnp.stack([g1u_lo, g1u_hi]), src_u2i, dst_u2i, z2d32)
    s1_user = _conv(jnp.stack([g1i_lo, g1i_hi]), src_i2u, dst_i2u, z2d32)

    o_user = _final(s1_user[0, :N], s1_user[1, :N], cnt_user, r1_user)
    o_item = _final(s1_item[0, :N], s1_item[1, :N], cnt_item, r1_item)
    return (o_user, o_item)
